# baseline (speedup 1.0000x reference)
import functools

import jax
import jax.numpy as jnp
from jax import lax
from jax.experimental import pallas as pl
from jax.experimental.pallas import tpu as pltpu
from jax.experimental.pallas import tpu_sc as plsc

N_USER = 10000
N_SPOT = 10000
E = 320000
D = 128

NPAD = 10240
NC = 2
NS = 16
CHUNK = 128
ROWS_PER_TILE = NPAD // NS
N_CHUNKS = 160
GRP = 8
NGRP = N_CHUNKS // GRP
EPAD = N_CHUNKS * NS * CHUNK
HCOLS = 16

GCHUNK = 64
GN_CHUNKS = EPAD // (NS * GCHUNK)
GGRP = 16
GNGRP = GN_CHUNKS // GGRP
NBUF = 4


def _hist_body(ids_ref, hist_out_ref, idx_v, ones_v, zrow_v, hist_sh):
  c = lax.axis_index("c")
  s = lax.axis_index("s")

  lane = lax.iota(jnp.int32, HCOLS)
  one_row = jnp.where(lane == 0, 1.0, 0.0).astype(jnp.float32)

  def init_rows(i, _):
    ones_v[i, :] = one_row
    zrow_v[i, :] = jnp.zeros((HCOLS,), jnp.float32)
    return 0

  lax.fori_loop(0, CHUNK, init_rows, 0)

  for k in range(ROWS_PER_TILE // CHUNK):
    pltpu.sync_copy(zrow_v, hist_sh.at[pl.ds(s * ROWS_PER_TILE + k * CHUNK, CHUNK)])

  plsc.subcore_barrier()

  def count(g, _):
    base = (c * NS + s) * N_CHUNKS + g * GRP
    pltpu.sync_copy(ids_ref.at[pl.ds(base, GRP)], idx_v)
    for j in range(GRP):
      pltpu.sync_copy(ones_v, hist_sh.at[idx_v.at[j]], add=True)
    return 0

  lax.fori_loop(0, NGRP, count, 0)
  plsc.subcore_barrier()

  for k in range(ROWS_PER_TILE // CHUNK):
    base = s * ROWS_PER_TILE + k * CHUNK
    pltpu.sync_copy(hist_sh.at[pl.ds(base, CHUNK)], zrow_v)
    pltpu.sync_copy(zrow_v, hist_out_ref.at[c, pl.ds(base, CHUNK)])


def _gather_scatter_body(src_ids_ref, dst_ids_ref, xn_ref, out_ref,
                         idx_src_v, idx_dst_v, rows0_v, rows1_v, rows2_v,
                         rows3_v, sem0, sem1, sem2, sem3, acc_sh):
  c = lax.axis_index("c")
  s = lax.axis_index("s")
  bufs = (rows0_v, rows1_v, rows2_v, rows3_v)
  sems = (sem0, sem1, sem2, sem3)

  def zero_rows(i, _):
    for k in range(D // 16):
      rows0_v[i, pl.ds(k * 16, 16)] = jnp.zeros((16,), jnp.float32)
    return 0

  lax.fori_loop(0, GCHUNK, zero_rows, 0)
  for k in range(ROWS_PER_TILE // GCHUNK):
    pltpu.sync_copy(rows0_v, acc_sh.at[pl.ds(s * ROWS_PER_TILE + k * GCHUNK, GCHUNK)])

  plsc.subcore_barrier()

  def group(g, _):
    base = (c * NS + s) * GN_CHUNKS + g * GGRP
    pltpu.sync_copy(src_ids_ref.at[pl.ds(base, GGRP)], idx_src_v)
    pltpu.sync_copy(dst_ids_ref.at[pl.ds(base, GGRP)], idx_dst_v)

    def fire(j):
      return pltpu.async_copy(
          xn_ref.at[idx_src_v.at[j]], bufs[j % NBUF], sems[j % NBUF])

    desc = {j: fire(j) for j in range(NBUF)}
    for j in range(GGRP):
      desc[j].wait()
      pltpu.sync_copy(bufs[j % NBUF], acc_sh.at[idx_dst_v.at[j]], add=True)
      if j + NBUF < GGRP:
        desc[j + NBUF] = fire(j + NBUF)
    return 0

  lax.fori_loop(0, GNGRP, group, 0)
  plsc.subcore_barrier()

  for k in range(ROWS_PER_TILE // GCHUNK):
    base = s * ROWS_PER_TILE + k * GCHUNK
    pltpu.sync_copy(acc_sh.at[pl.ds(base, GCHUNK)], rows0_v)
    pltpu.sync_copy(rows0_v, out_ref.at[c, pl.ds(base, GCHUNK)])


def _scale_body(x_ref, h_ref, o_ref):
  h = h_ref[...]
  div = jnp.where(h == 0.0, 1e-06, h)
  o_ref[...] = x_ref[...] * lax.rsqrt(div)


def _scale_rows(x, h_col):
  rows = x.shape[0]
  blk = 256
  return pl.pallas_call(
      _scale_body,
      grid=(rows // blk,),
      in_specs=[
          pl.BlockSpec((blk, D), lambda i: (i, 0)),
          pl.BlockSpec((blk, 1), lambda i: (i, 0)),
      ],
      out_specs=pl.BlockSpec((blk, D), lambda i: (i, 0)),
      out_shape=jax.ShapeDtypeStruct((rows, D), jnp.float32),
  )(x, h_col)


@jax.jit
def kernel(user_x, spot_x, user_spot):
  mesh = plsc.VectorSubcoreMesh(
      core_axis_name="c", subcore_axis_name="s", num_cores=NC, num_subcores=NS)

  ids = user_spot.astype(jnp.int32)
  pad = jnp.full((2, EPAD - E), NPAD - 1, jnp.int32)
  ids_pad = jnp.concatenate([ids, pad], axis=1)
  ids_r = ids_pad.reshape(2 * NS * N_CHUNKS, CHUNK)

  src_ids = jnp.stack([ids_pad[1], ids_pad[0] + NPAD])
  src_ids_r = src_ids.reshape(2 * NS * GN_CHUNKS, GCHUNK)
  dst_ids_r = ids_pad.reshape(2 * NS * GN_CHUNKS, GCHUNK)

  hist_kernel = pl.kernel(
      _hist_body,
      out_type=jax.ShapeDtypeStruct((2, NPAD, HCOLS), jnp.float32),
      mesh=mesh,
      scratch_types=[
          pltpu.VMEM((GRP, CHUNK), jnp.int32),
          pltpu.VMEM((CHUNK, HCOLS), jnp.float32),
          pltpu.VMEM((CHUNK, HCOLS), jnp.float32),
          pltpu.VMEM_SHARED((NPAD, HCOLS), jnp.float32),
      ],
  )
  hist = hist_kernel(ids_r)
  hu = hist[0, :, 0:1]
  hs = hist[1, :, 0:1]

  user_x_pad = jnp.zeros((NPAD, D), jnp.float32).at[:N_USER].set(user_x)
  spot_x_pad = jnp.zeros((NPAD, D), jnp.float32).at[:N_SPOT].set(spot_x)

  x_cat = jnp.concatenate([spot_x_pad, user_x_pad], axis=0)
  h_cat = jnp.concatenate([hs, hu], axis=0)
  xn_cat = _scale_rows(x_cat, h_cat)

  gs_kernel = pl.kernel(
      _gather_scatter_body,
      out_type=jax.ShapeDtypeStruct((2, NPAD, D), jnp.float32),
      mesh=mesh,
      scratch_types=[
          pltpu.VMEM((GGRP, GCHUNK), jnp.int32),
          pltpu.VMEM((GGRP, GCHUNK), jnp.int32),
          pltpu.VMEM((GCHUNK, D), jnp.float32),
          pltpu.VMEM((GCHUNK, D), jnp.float32),
          pltpu.VMEM((GCHUNK, D), jnp.float32),
          pltpu.VMEM((GCHUNK, D), jnp.float32),
          pltpu.SemaphoreType.DMA,
          pltpu.SemaphoreType.DMA,
          pltpu.SemaphoreType.DMA,
          pltpu.SemaphoreType.DMA,
          pltpu.VMEM_SHARED((NPAD, D), jnp.float32),
      ],
  )
  acc = gs_kernel(src_ids_r, dst_ids_r, xn_cat)

  acc_flat = acc.reshape(2 * NPAD, D)
  h_out = jnp.concatenate([hu, hs], axis=0)
  out_flat = _scale_rows(acc_flat, h_out)
  user_out = out_flat[:N_USER]
  spot_out = out_flat[NPAD:NPAD + N_SPOT]
  return (user_out, spot_out)


# speedup vs baseline: 1.0874x; 1.0874x over previous
import functools

import jax
import jax.numpy as jnp
from jax import lax
from jax.experimental import pallas as pl
from jax.experimental.pallas import tpu as pltpu
from jax.experimental.pallas import tpu_sc as plsc

N_USER = 10000
N_SPOT = 10000
E = 320000
D = 128

NPAD = 10240
NC = 2
NS = 16
CHUNK = 128
ROWS_PER_TILE = NPAD // NS
N_CHUNKS = 160
GRP = 8
NGRP = N_CHUNKS // GRP
EPAD = N_CHUNKS * NS * CHUNK
HCOLS = 16

GCHUNK = 128
GN_CHUNKS = EPAD // (NS * GCHUNK)
GGRP = 8
GNGRP = GN_CHUNKS // GGRP
NBUF = 2


def _hist_body(ids_ref, hist_out_ref, idx_v, ones_v, zrow_v, hist_sh):
  c = lax.axis_index("c")
  s = lax.axis_index("s")

  lane = lax.iota(jnp.int32, HCOLS)
  one_row = jnp.where(lane == 0, 1.0, 0.0).astype(jnp.float32)

  def init_rows(i, _):
    ones_v[i, :] = one_row
    zrow_v[i, :] = jnp.zeros((HCOLS,), jnp.float32)
    return 0

  lax.fori_loop(0, CHUNK, init_rows, 0)

  for k in range(ROWS_PER_TILE // CHUNK):
    pltpu.sync_copy(zrow_v, hist_sh.at[pl.ds(s * ROWS_PER_TILE + k * CHUNK, CHUNK)])

  plsc.subcore_barrier()

  def count(g, _):
    base = (c * NS + s) * N_CHUNKS + g * GRP
    pltpu.sync_copy(ids_ref.at[pl.ds(base, GRP)], idx_v)
    for j in range(GRP):
      pltpu.sync_copy(ones_v, hist_sh.at[idx_v.at[j]], add=True)
    return 0

  lax.fori_loop(0, NGRP, count, 0)
  plsc.subcore_barrier()

  for k in range(ROWS_PER_TILE // CHUNK):
    base = s * ROWS_PER_TILE + k * CHUNK
    pltpu.sync_copy(hist_sh.at[pl.ds(base, CHUNK)], zrow_v)
    pltpu.sync_copy(zrow_v, hist_out_ref.at[c, pl.ds(base, CHUNK)])


def _gather_scatter_body(src_ids_ref, dst_ids_ref, xn_ref, wdst_ref, out_ref,
                         idx_src_v, idx_dst_v, rows0_v, rows1_v, w_v,
                         sem0, sem1, acc_sh):
  c = lax.axis_index("c")
  s = lax.axis_index("s")
  bufs = (rows0_v, rows1_v)
  sems = (sem0, sem1)

  def zero_rows(i, _):
    for k in range(D // 16):
      rows0_v[i, pl.ds(k * 16, 16)] = jnp.zeros((16,), jnp.float32)
    return 0

  lax.fori_loop(0, GCHUNK, zero_rows, 0)
  for k in range(ROWS_PER_TILE // GCHUNK):
    pltpu.sync_copy(rows0_v, acc_sh.at[pl.ds(s * ROWS_PER_TILE + k * GCHUNK, GCHUNK)])

  plsc.subcore_barrier()

  def group(g, _):
    base = (c * NS + s) * GN_CHUNKS + g * GGRP
    pltpu.sync_copy(src_ids_ref.at[pl.ds(base, GGRP)], idx_src_v)
    pltpu.sync_copy(dst_ids_ref.at[pl.ds(base, GGRP)], idx_dst_v)

    def fire(j):
      return pltpu.async_copy(
          xn_ref.at[idx_src_v.at[j]], bufs[j % NBUF], sems[j % NBUF])

    desc = {j: fire(j) for j in range(NBUF)}
    for j in range(GGRP):
      desc[j].wait()
      pltpu.sync_copy(bufs[j % NBUF], acc_sh.at[idx_dst_v.at[j]], add=True)
      if j + NBUF < GGRP:
        desc[j + NBUF] = fire(j + NBUF)
    return 0

  lax.fori_loop(0, GNGRP, group, 0)

  pltpu.sync_copy(
      wdst_ref.at[pl.ds(c * NPAD + s * ROWS_PER_TILE, ROWS_PER_TILE)], w_v)
  plsc.subcore_barrier()

  for k in range(ROWS_PER_TILE // GCHUNK):
    base = s * ROWS_PER_TILE + k * GCHUNK
    pltpu.sync_copy(acc_sh.at[pl.ds(base, GCHUNK)], rows0_v)

    def scale_grp(i16, _, k=k):
      wv = w_v[pl.ds(k * GCHUNK + i16 * 16, 16)]
      for r in range(16):
        w = wv[r]
        for m in range(D // 16):
          i = i16 * 16 + r
          rows0_v[i, pl.ds(m * 16, 16)] = rows0_v[i, pl.ds(m * 16, 16)] * w
      return 0

    lax.fori_loop(0, GCHUNK // 16, scale_grp, 0)
    pltpu.sync_copy(rows0_v, out_ref.at[c, pl.ds(base, GCHUNK)])


def _scale_body(x_ref, h_ref, o_ref, w_ref):
  h = h_ref[...]
  div = jnp.where(h == 0.0, 1e-06, h)
  w = lax.rsqrt(div)
  w_ref[...] = w
  o_ref[...] = x_ref[...] * w


def _scale_rows(x, h_col):
  rows = x.shape[0]
  blk = 256
  return pl.pallas_call(
      _scale_body,
      grid=(rows // blk,),
      in_specs=[
          pl.BlockSpec((blk, D), lambda i: (i, 0)),
          pl.BlockSpec((blk, 1), lambda i: (i, 0)),
      ],
      out_specs=[
          pl.BlockSpec((blk, D), lambda i: (i, 0)),
          pl.BlockSpec((blk, 1), lambda i: (i, 0)),
      ],
      out_shape=[
          jax.ShapeDtypeStruct((rows, D), jnp.float32),
          jax.ShapeDtypeStruct((rows, 1), jnp.float32),
      ],
  )(x, h_col)


@jax.jit
def kernel(user_x, spot_x, user_spot):
  mesh = plsc.VectorSubcoreMesh(
      core_axis_name="c", subcore_axis_name="s", num_cores=NC, num_subcores=NS)

  ids = user_spot.astype(jnp.int32)
  pad = jnp.full((2, EPAD - E), NPAD - 1, jnp.int32)
  ids_pad = jnp.concatenate([ids, pad], axis=1)
  ids_r = ids_pad.reshape(2 * NS * N_CHUNKS, CHUNK)

  src_ids = jnp.stack([ids_pad[1], ids_pad[0] + NPAD])
  src_ids_r = src_ids.reshape(2 * NS * GN_CHUNKS, GCHUNK)
  dst_ids_r = ids_pad.reshape(2 * NS * GN_CHUNKS, GCHUNK)

  hist_kernel = pl.kernel(
      _hist_body,
      out_type=jax.ShapeDtypeStruct((2, NPAD, HCOLS), jnp.float32),
      mesh=mesh,
      scratch_types=[
          pltpu.VMEM((GRP, CHUNK), jnp.int32),
          pltpu.VMEM((CHUNK, HCOLS), jnp.float32),
          pltpu.VMEM((CHUNK, HCOLS), jnp.float32),
          pltpu.VMEM_SHARED((NPAD, HCOLS), jnp.float32),
      ],
  )
  hist = hist_kernel(ids_r)
  hu = hist[0, :, 0:1]
  hs = hist[1, :, 0:1]

  user_x_pad = jnp.zeros((NPAD, D), jnp.float32).at[:N_USER].set(user_x)
  spot_x_pad = jnp.zeros((NPAD, D), jnp.float32).at[:N_SPOT].set(spot_x)

  x_cat = jnp.concatenate([spot_x_pad, user_x_pad], axis=0)
  h_cat = jnp.concatenate([hs, hu], axis=0)
  xn_cat, w_cat = _scale_rows(x_cat, h_cat)
  wdst = jnp.concatenate([w_cat[NPAD:], w_cat[:NPAD]]).reshape(2 * NPAD)

  gs_kernel = pl.kernel(
      _gather_scatter_body,
      out_type=jax.ShapeDtypeStruct((2, NPAD, D), jnp.float32),
      mesh=mesh,
      scratch_types=[
          pltpu.VMEM((GGRP, GCHUNK), jnp.int32),
          pltpu.VMEM((GGRP, GCHUNK), jnp.int32),
          pltpu.VMEM((GCHUNK, D), jnp.float32),
          pltpu.VMEM((GCHUNK, D), jnp.float32),
          pltpu.VMEM((ROWS_PER_TILE,), jnp.float32),
          pltpu.SemaphoreType.DMA,
          pltpu.SemaphoreType.DMA,
          pltpu.VMEM_SHARED((NPAD, D), jnp.float32),
      ],
  )
  out = gs_kernel(src_ids_r, dst_ids_r, xn_cat, wdst)
  return (out[0, :N_USER], out[1, :N_SPOT])
